# baseline (speedup 1.0000x reference)
import functools

import jax
import jax.numpy as jnp
from jax import lax
from jax.experimental import pallas as pl
from jax.experimental.pallas import tpu as pltpu
from jax.experimental.pallas import tpu_sc as plsc

B = 4096
D = 64
K = 10
BM = 256
BN = 256
NB = B // BM
NEG = -1e30

NC = 2
NS = 16
NW = NC * NS
RPW = B // NW
RG = 8
NL = 16


def _sim_lse_kernel(img_ref, txt_ref, imgT_ref, txtT_ref, scale_ref,
                    sim_out, lsei_out, lset_out, lii_out, simd_out):
    i = pl.program_id(0)
    r0 = i * BM
    a_img = img_ref[pl.ds(r0, BM), :]
    a_txt = txt_ref[pl.ds(r0, BM), :]
    scale = scale_ref[0, 0]

    ns_i = jnp.sum(a_txt * a_txt, axis=1, keepdims=True)
    n_i = jnp.sqrt(ns_i)
    row_ids = r0 + lax.broadcasted_iota(jnp.int32, (BM, BN), 0)
    loc_col = lax.broadcasted_iota(jnp.int32, (BM, BN), 1)
    dims = (((1,), (0,)), ((), ()))

    m_i = jnp.full((BM, 1), NEG, jnp.float32)
    s_i = jnp.zeros((BM, 1), jnp.float32)
    m_t = jnp.full((BM, 1), NEG, jnp.float32)
    s_t = jnp.zeros((BM, 1), jnp.float32)

    for j in range(NB):
        c0 = j * BN
        t_j = txtT_ref[:, pl.ds(c0, BN)]
        i_j = imgT_ref[:, pl.ds(c0, BN)]
        dot = lax.dot_general(a_txt, t_j, dims,
                              preferred_element_type=jnp.float32)
        n_j = jnp.sqrt(jnp.sum(t_j * t_j, axis=0, keepdims=True))
        sim = dot / jnp.maximum(n_i * n_j, 1e-8)
        sim = jnp.where(row_ids == c0 + loc_col, 0.0, sim)
        sim_out[:, pl.ds(c0, BN)] = sim.astype(jnp.bfloat16)

        lb = scale * lax.dot_general(a_img, t_j, dims,
                                     preferred_element_type=jnp.float32)
        ltb = scale * lax.dot_general(a_txt, i_j, dims,
                                      preferred_element_type=jnp.float32)
        bmax = jnp.max(lb, axis=1, keepdims=True)
        m_i2 = jnp.maximum(m_i, bmax)
        s_i = s_i * jnp.exp(m_i - m_i2) + jnp.sum(
            jnp.exp(lb - m_i2), axis=1, keepdims=True)
        m_i = m_i2
        bmax = jnp.max(ltb, axis=1, keepdims=True)
        m_t2 = jnp.maximum(m_t, bmax)
        s_t = s_t * jnp.exp(m_t - m_t2) + jnp.sum(
            jnp.exp(ltb - m_t2), axis=1, keepdims=True)
        m_t = m_t2

    lsei_out[...] = m_i + jnp.log(s_i)
    lset_out[...] = m_t + jnp.log(s_t)
    lii_out[...] = scale * jnp.sum(a_img * a_txt, axis=1, keepdims=True)
    simd_out[...] = ns_i / jnp.maximum(ns_i, 1e-8)


def _topk_sc(sim_hbm, iota_hbm, out_hbm, rowbuf, resbuf, iotabuf):
    wid = lax.axis_index("s") * NC + lax.axis_index("c")
    base = wid * RPW
    pltpu.sync_copy(iota_hbm, iotabuf)
    iota_v = iotabuf[...]

    def merge(acc, x):
        xs = plsc.sort_key_val(x, iota_v)[0]
        m = jnp.maximum(acc, lax.rev(xs, (0,)))
        return plsc.sort_key_val(m, iota_v)[0]

    def unpk(x):
        a = plsc.bitcast(lax.shift_left(x, 16), jnp.float32)
        b = plsc.bitcast(jnp.bitwise_and(x, jnp.int32(-65536)), jnp.float32)
        return a, b

    def group(gi, _):
        g0 = base + gi * RG
        pltpu.sync_copy(sim_hbm.at[pl.ds(g0, RG), :], rowbuf)

        inits = []
        for rr in range(RG):
            a, b = unpk(rowbuf[rr, pl.ds(0, NL)])
            inits.append(merge(plsc.sort_key_val(a, iota_v)[0], b))

        def scan(c, Rs):
            off = pl.multiple_of(c * NL, NL)
            new = []
            for rr in range(RG):
                a, b = unpk(rowbuf[rr, pl.ds(off, NL)])
                new.append(merge(merge(Rs[rr], a), b))
            return tuple(new)

        Rs = lax.fori_loop(1, (B // 2) // NL, scan, tuple(inits))
        for rr in range(RG):
            resbuf[rr, pl.ds(0, NL)] = Rs[rr]
        pltpu.sync_copy(resbuf, out_hbm.at[pl.ds(g0, RG), :])
        return 0

    lax.fori_loop(0, RPW // RG, group, 0)


def _select_kernel(img_ref, txt_ref, imgT_ref, txtT_ref, idxc_ref, idxr_ref,
                   scale_ref, th_ref, lsei_ref, lset_ref, lii_ref, simd_ref,
                   out_ref):
    i = pl.program_id(0)
    r0 = i * BM
    a_img = img_ref[pl.ds(r0, BM), :]
    a_txt = txt_ref[pl.ds(r0, BM), :]
    scale = scale_ref[0, 0]
    c_i = idxc_ref[pl.ds(r0, BM), :]
    thresh = th_ref[pl.ds(r0, BM), :]

    ns_i = jnp.sum(a_txt * a_txt, axis=1, keepdims=True)
    n_i = jnp.sqrt(ns_i)
    row_ids = r0 + lax.broadcasted_iota(jnp.int32, (BM, BN), 0)
    loc_col = lax.broadcasted_iota(jnp.int32, (BM, BN), 1)
    dims = (((1,), (0,)), ((), ()))

    def body(j, carry):
        rs, ws = carry
        c0 = j * BN
        t_j = txtT_ref[:, pl.ds(c0, BN)]
        i_j = imgT_ref[:, pl.ds(c0, BN)]
        dot = lax.dot_general(a_txt, t_j, dims,
                              preferred_element_type=jnp.float32)
        n_j = jnp.sqrt(jnp.sum(t_j * t_j, axis=0, keepdims=True))
        sim = dot / jnp.maximum(n_i * n_j, 1e-8)
        sim = jnp.where(row_ids == c0 + loc_col, 0.0, sim)
        lb = scale * lax.dot_general(a_img, t_j, dims,
                                     preferred_element_type=jnp.float32)
        ltb = scale * lax.dot_general(a_txt, i_j, dims,
                                      preferred_element_type=jnp.float32)
        c_j = idxr_ref[:, pl.ds(c0, BN)]
        w = jnp.where((sim >= thresh) & (c_i == c_j), sim, 0.0)
        rs = rs + jnp.sum(w, axis=1, keepdims=True)
        ws = ws + jnp.sum(w * (lb + ltb), axis=1, keepdims=True)
        return rs, ws

    rs, ws = lax.fori_loop(
        0, NB, body,
        (jnp.zeros((BM, 1), jnp.float32), jnp.zeros((BM, 1), jnp.float32)))

    simd = simd_ref[pl.ds(r0, BM), :]
    lii = lii_ref[pl.ds(r0, BM), :]
    rowsum = simd + rs
    wsum = simd * 2.0 * lii + ws
    out_ref[...] = (lsei_ref[pl.ds(r0, BM), :] + lset_ref[pl.ds(r0, BM), :]
                    - wsum / rowsum)


@functools.partial(jax.jit, static_argnames=("interpret",))
def kernel(image_features, text_features, logit_scale, img_index,
           interpret=False):
    img = image_features.astype(jnp.float32)
    txt = text_features.astype(jnp.float32)
    scale = jnp.asarray(logit_scale, jnp.float32).reshape(1, 1)
    idxc = img_index.astype(jnp.int32).reshape(B, 1)
    idxr = img_index.astype(jnp.int32).reshape(1, B)

    full = lambda shape: pl.BlockSpec(shape, lambda i: (0,) * len(shape))
    colspec = pl.BlockSpec((BM, 1), lambda i: (i, 0))

    sim, lse_i, lse_t, lii, simd = pl.pallas_call(
        _sim_lse_kernel,
        grid=(NB,),
        in_specs=[full((B, D)), full((B, D)), full((D, B)), full((D, B)),
                  full((1, 1))],
        out_specs=[pl.BlockSpec((BM, B), lambda i: (i, 0)),
                   colspec, colspec, colspec, colspec],
        out_shape=[jax.ShapeDtypeStruct((B, B), jnp.bfloat16)] +
                  [jax.ShapeDtypeStruct((B, 1), jnp.float32)] * 4,
        interpret=interpret,
    )(img, txt, img.T, txt.T, scale)

    sim_i32 = lax.bitcast_convert_type(
        sim.reshape(B, B // 2, 2), jnp.int32)
    iota16 = jnp.arange(NL, dtype=jnp.int32)
    mesh = plsc.VectorSubcoreMesh(core_axis_name="c", subcore_axis_name="s")
    top16 = pl.kernel(
        _topk_sc,
        out_type=jax.ShapeDtypeStruct((B, NL), jnp.float32),
        mesh=mesh,
        scratch_types=[pltpu.VMEM((RG, B // 2), jnp.int32),
                       pltpu.VMEM((RG, NL), jnp.float32),
                       pltpu.VMEM((NL,), jnp.int32)],
        compiler_params=pltpu.CompilerParams(needs_layout_passes=False),
        interpret=interpret,
    )(sim_i32, iota16)
    thresh = top16[:, 6:7]

    out = pl.pallas_call(
        _select_kernel,
        grid=(NB,),
        in_specs=[full((B, D)), full((B, D)), full((D, B)), full((D, B)),
                  full((B, 1)), full((1, B)), full((1, 1)), full((B, 1)),
                  full((B, 1)), full((B, 1)), full((B, 1)), full((B, 1))],
        out_specs=colspec,
        out_shape=jax.ShapeDtypeStruct((B, 1), jnp.float32),
        interpret=interpret,
    )(img, txt, img.T, txt.T, idxc, idxr, scale, thresh,
      lse_i, lse_t, lii, simd)
    return 0.5 * jnp.mean(out)


# speedup vs baseline: 2.1482x; 2.1482x over previous
import functools

import jax
import jax.numpy as jnp
from jax import lax
from jax.experimental import pallas as pl
from jax.experimental.pallas import tpu as pltpu
from jax.experimental.pallas import tpu_sc as plsc

B = 4096
D = 64
K = 10
BM = 256
BN = 256
NB = B // BM
NEG = -1e30

NC = 2
NS = 16
NW = NC * NS
RPW = B // NW
RG = 8
NL = 16


def _sim_lse_kernel(img_ref, txt_ref, imgT_ref, txtT_ref, scale_ref,
                    sim_out, lsei_out, lset_out, lii_out, simd_out):
    i = pl.program_id(0)
    r0 = i * BM
    a_img = img_ref[pl.ds(r0, BM), :]
    a_txt = txt_ref[pl.ds(r0, BM), :]
    scale = scale_ref[0, 0]

    ns_i = jnp.sum(a_txt * a_txt, axis=1, keepdims=True)
    n_i = jnp.sqrt(ns_i)
    row_ids = r0 + lax.broadcasted_iota(jnp.int32, (BM, BN), 0)
    loc_col = lax.broadcasted_iota(jnp.int32, (BM, BN), 1)
    dims = (((1,), (0,)), ((), ()))

    m_i = jnp.full((BM, 1), NEG, jnp.float32)
    s_i = jnp.zeros((BM, 1), jnp.float32)
    m_t = jnp.full((BM, 1), NEG, jnp.float32)
    s_t = jnp.zeros((BM, 1), jnp.float32)

    for j in range(NB):
        c0 = j * BN
        t_j = txtT_ref[:, pl.ds(c0, BN)]
        i_j = imgT_ref[:, pl.ds(c0, BN)]
        dot = lax.dot_general(a_txt, t_j, dims,
                              preferred_element_type=jnp.float32)
        n_j = jnp.sqrt(jnp.sum(t_j * t_j, axis=0, keepdims=True))
        sim = dot / jnp.maximum(n_i * n_j, 1e-8)
        sim = jnp.where(row_ids == c0 + loc_col, 0.0, sim)
        sim_out[:, pl.ds(c0, BN)] = sim

        lb = scale * lax.dot_general(a_img, t_j, dims,
                                     preferred_element_type=jnp.float32)
        ltb = scale * lax.dot_general(a_txt, i_j, dims,
                                      preferred_element_type=jnp.float32)
        bmax = jnp.max(lb, axis=1, keepdims=True)
        m_i2 = jnp.maximum(m_i, bmax)
        s_i = s_i * jnp.exp(m_i - m_i2) + jnp.sum(
            jnp.exp(lb - m_i2), axis=1, keepdims=True)
        m_i = m_i2
        bmax = jnp.max(ltb, axis=1, keepdims=True)
        m_t2 = jnp.maximum(m_t, bmax)
        s_t = s_t * jnp.exp(m_t - m_t2) + jnp.sum(
            jnp.exp(ltb - m_t2), axis=1, keepdims=True)
        m_t = m_t2

    lsei_out[...] = m_i + jnp.log(s_i)
    lset_out[...] = m_t + jnp.log(s_t)
    lii_out[...] = scale * jnp.sum(a_img * a_txt, axis=1, keepdims=True)
    simd_out[...] = ns_i / jnp.maximum(ns_i, 1e-8)


def _topk_sc(sim_hbm, iota_hbm, out_hbm, rowbuf, resbuf, iotabuf):
    wid = lax.axis_index("s") * NC + lax.axis_index("c")
    base = wid * RPW
    pltpu.sync_copy(iota_hbm, iotabuf)
    iota_v = iotabuf[...]

    def group(gi, _):
        g0 = base + gi * RG
        pltpu.sync_copy(sim_hbm.at[pl.ds(g0, RG), :], rowbuf)

        inits = []
        for rr in range(RG):
            g = g0 + rr
            cd = lax.shift_right_logical(g, 4)
            off = pl.multiple_of(lax.shift_left(cd, 4), NL)
            xv = rowbuf[rr, pl.ds(off, NL)]
            xv = jnp.where(iota_v == jnp.bitwise_and(g, 15), 0.0, xv)
            rowbuf[rr, pl.ds(off, NL)] = xv
        for rr in range(RG):
            inits.append(plsc.sort_key_val(rowbuf[rr, pl.ds(0, NL)], iota_v)[0])

        def scan(c, Rs):
            off = pl.multiple_of(c * NL, NL)
            new = []
            for rr in range(RG):
                x = plsc.sort_key_val(rowbuf[rr, pl.ds(off, NL)], iota_v)[0]
                rx = lax.rev(x, (0,))
                merged = jnp.maximum(Rs[rr], rx)
                new.append(plsc.sort_key_val(merged, iota_v)[0])
            return tuple(new)

        Rs = lax.fori_loop(1, B // NL, scan, tuple(inits))
        for rr in range(RG):
            resbuf[rr, pl.ds(0, NL)] = Rs[rr]
        pltpu.sync_copy(resbuf, out_hbm.at[pl.ds(g0, RG), :])
        return 0

    lax.fori_loop(0, RPW // RG, group, 0)


def _select_kernel(sim_ref, img_ref, txt_ref, imgT_ref, txtT_ref, idxc_ref,
                   idxr_ref, scale_ref, th_ref, lsei_ref, lset_ref, lii_ref,
                   simd_ref, out_ref):
    i = pl.program_id(0)
    r0 = i * BM
    a_img = img_ref[pl.ds(r0, BM), :]
    a_txt = txt_ref[pl.ds(r0, BM), :]
    scale = scale_ref[0, 0]
    c_i = idxc_ref[pl.ds(r0, BM), :]
    thresh = th_ref[pl.ds(r0, BM), :]

    dims = (((1,), (0,)), ((), ()))

    def body(j, carry):
        rs, ws = carry
        c0 = j * BN
        t_j = txtT_ref[:, pl.ds(c0, BN)]
        i_j = imgT_ref[:, pl.ds(c0, BN)]
        sim = sim_ref[:, pl.ds(c0, BN)]
        lb = scale * lax.dot_general(a_img, t_j, dims,
                                     preferred_element_type=jnp.float32)
        ltb = scale * lax.dot_general(a_txt, i_j, dims,
                                      preferred_element_type=jnp.float32)
        c_j = idxr_ref[:, pl.ds(c0, BN)]
        w = jnp.where((sim >= thresh) & (c_i == c_j), sim, 0.0)
        rs = rs + jnp.sum(w, axis=1, keepdims=True)
        ws = ws + jnp.sum(w * (lb + ltb), axis=1, keepdims=True)
        return rs, ws

    rs, ws = lax.fori_loop(
        0, NB, body,
        (jnp.zeros((BM, 1), jnp.float32), jnp.zeros((BM, 1), jnp.float32)))

    simd = simd_ref[pl.ds(r0, BM), :]
    lii = lii_ref[pl.ds(r0, BM), :]
    rowsum = simd + rs
    wsum = simd * 2.0 * lii + ws
    out_ref[...] = (lsei_ref[pl.ds(r0, BM), :] + lset_ref[pl.ds(r0, BM), :]
                    - wsum / rowsum)


@functools.partial(jax.jit, static_argnames=("interpret",))
def kernel(image_features, text_features, logit_scale, img_index,
           interpret=False):
    img = image_features.astype(jnp.float32)
    txt = text_features.astype(jnp.float32)
    scale = jnp.asarray(logit_scale, jnp.float32).reshape(1, 1)
    idxc = img_index.astype(jnp.int32).reshape(B, 1)
    idxr = img_index.astype(jnp.int32).reshape(1, B)

    full = lambda shape: pl.BlockSpec(shape, lambda i: (0,) * len(shape))
    colspec = pl.BlockSpec((BM, 1), lambda i: (i, 0))

    sim, lse_i, lse_t, lii, simd = pl.pallas_call(
        _sim_lse_kernel,
        grid=(NB,),
        in_specs=[full((B, D)), full((B, D)), full((D, B)), full((D, B)),
                  full((1, 1))],
        out_specs=[pl.BlockSpec((BM, B), lambda i: (i, 0)),
                   colspec, colspec, colspec, colspec],
        out_shape=[jax.ShapeDtypeStruct((B, B), jnp.float32)] +
                  [jax.ShapeDtypeStruct((B, 1), jnp.float32)] * 4,
        interpret=interpret,
    )(img, txt, img.T, txt.T, scale)

    iota16 = jnp.arange(NL, dtype=jnp.int32)
    mesh = plsc.VectorSubcoreMesh(core_axis_name="c", subcore_axis_name="s")
    top16 = pl.kernel(
        _topk_sc,
        out_type=jax.ShapeDtypeStruct((B, NL), jnp.float32),
        mesh=mesh,
        scratch_types=[pltpu.VMEM((RG, B), jnp.float32),
                       pltpu.VMEM((RG, NL), jnp.float32),
                       pltpu.VMEM((NL,), jnp.int32)],
        compiler_params=pltpu.CompilerParams(needs_layout_passes=False),
        interpret=interpret,
    )(sim, iota16)
    thresh = top16[:, 6:7]

    out = pl.pallas_call(
        _select_kernel,
        grid=(NB,),
        in_specs=[pl.BlockSpec((BM, B), lambda i: (i, 0)),
                  full((B, D)), full((B, D)), full((D, B)), full((D, B)),
                  full((B, 1)), full((1, B)), full((1, 1)), full((B, 1)),
                  full((B, 1)), full((B, 1)), full((B, 1)), full((B, 1))],
        out_specs=colspec,
        out_shape=jax.ShapeDtypeStruct((B, 1), jnp.float32),
        interpret=interpret,
    )(sim, img, txt, img.T, txt.T, idxc, idxr, scale, thresh,
      lse_i, lse_t, lii, simd)
    return 0.5 * jnp.mean(out)


# speedup vs baseline: 2.6452x; 1.2314x over previous
import functools

import jax
import jax.numpy as jnp
from jax import lax
from jax.experimental import pallas as pl
from jax.experimental.pallas import tpu as pltpu
from jax.experimental.pallas import tpu_sc as plsc

B = 4096
D = 64
K = 10
BM = 256
BN = 256
NB = B // BM
NEG = -1e30

NCH = 2
CH = B // NCH
NBC = CH // BM

NC = 2
NS = 16
NW = NC * NS
RPW = CH // NW
RG = 8
NL = 16


def _make_sim_lse(q):
    qoff = q * CH

    def body(img_ref, txt_ref, imgT_ref, txtT_ref, scale_ref,
             sim_out, lsei_out, lset_out, lii_out, simd_out):
        i = pl.program_id(0)
        r0 = qoff + i * BM
        a_img = img_ref[pl.ds(r0, BM), :]
        a_txt = txt_ref[pl.ds(r0, BM), :]
        scale = scale_ref[0, 0]

        ns_i = jnp.sum(a_txt * a_txt, axis=1, keepdims=True)
        n_i = jnp.sqrt(ns_i)
        row_ids = r0 + lax.broadcasted_iota(jnp.int32, (BM, BN), 0)
        loc_col = lax.broadcasted_iota(jnp.int32, (BM, BN), 1)
        dims = (((1,), (0,)), ((), ()))

        m_i = jnp.full((BM, 1), NEG, jnp.float32)
        s_i = jnp.zeros((BM, 1), jnp.float32)
        m_t = jnp.full((BM, 1), NEG, jnp.float32)
        s_t = jnp.zeros((BM, 1), jnp.float32)

        for j in range(NB):
            c0 = j * BN
            t_j = txtT_ref[:, pl.ds(c0, BN)]
            i_j = imgT_ref[:, pl.ds(c0, BN)]
            dot = lax.dot_general(a_txt, t_j, dims,
                                  preferred_element_type=jnp.float32)
            n_j = jnp.sqrt(jnp.sum(t_j * t_j, axis=0, keepdims=True))
            sim = dot / jnp.maximum(n_i * n_j, 1e-8)
            sim = jnp.where(row_ids == c0 + loc_col, 0.0, sim)
            sim_out[:, pl.ds(c0, BN)] = sim

            lb = scale * lax.dot_general(a_img, t_j, dims,
                                         preferred_element_type=jnp.float32)
            ltb = scale * lax.dot_general(a_txt, i_j, dims,
                                          preferred_element_type=jnp.float32)
            bmax = jnp.max(lb, axis=1, keepdims=True)
            m_i2 = jnp.maximum(m_i, bmax)
            s_i = s_i * jnp.exp(m_i - m_i2) + jnp.sum(
                jnp.exp(lb - m_i2), axis=1, keepdims=True)
            m_i = m_i2
            bmax = jnp.max(ltb, axis=1, keepdims=True)
            m_t2 = jnp.maximum(m_t, bmax)
            s_t = s_t * jnp.exp(m_t - m_t2) + jnp.sum(
                jnp.exp(ltb - m_t2), axis=1, keepdims=True)
            m_t = m_t2

        lsei_out[...] = m_i + jnp.log(s_i)
        lset_out[...] = m_t + jnp.log(s_t)
        lii_out[...] = scale * jnp.sum(a_img * a_txt, axis=1, keepdims=True)
        simd_out[...] = ns_i / jnp.maximum(ns_i, 1e-8)

    return body


def _topk_sc(sim_hbm, iota_hbm, out_hbm, rowbuf, resbuf, iotabuf):
    wid = lax.axis_index("s") * NC + lax.axis_index("c")
    base = wid * RPW
    pltpu.sync_copy(iota_hbm, iotabuf)
    iota_v = iotabuf[...]

    def merge(acc, x):
        xs = plsc.sort_key_val(x, iota_v)[0]
        m = jnp.maximum(acc, lax.rev(xs, (0,)))
        return plsc.sort_key_val(m, iota_v)[0]

    def group(gi, _):
        g0 = base + gi * RG
        pltpu.sync_copy(sim_hbm.at[pl.ds(g0, RG), :], rowbuf)
        inits = []
        for rr in range(RG):
            inits.append(
                plsc.sort_key_val(rowbuf[rr, pl.ds(0, NL)], iota_v)[0])

        def scan(c, Rs):
            off = pl.multiple_of(c * NL, NL)
            new = []
            for rr in range(RG):
                new.append(merge(Rs[rr], rowbuf[rr, pl.ds(off, NL)]))
            return tuple(new)

        Rs = lax.fori_loop(1, B // NL, scan, tuple(inits))
        for rr in range(RG):
            resbuf[rr, pl.ds(0, NL)] = Rs[rr]
        pltpu.sync_copy(resbuf, out_hbm.at[pl.ds(g0, RG), :])
        return 0

    lax.fori_loop(0, RPW // RG, group, 0)


def _make_select(q):
    qoff = q * CH

    def body(sim_ref, img_ref, txt_ref, imgT_ref, txtT_ref, idxc_ref,
             idxr_ref, scale_ref, th_ref, lsei_ref, lset_ref, lii_ref,
             simd_ref, out_ref):
        i = pl.program_id(0)
        r0 = qoff + i * BM
        rl = i * BM
        a_img = img_ref[pl.ds(r0, BM), :]
        a_txt = txt_ref[pl.ds(r0, BM), :]
        scale = scale_ref[0, 0]
        c_i = idxc_ref[pl.ds(r0, BM), :]
        thresh = th_ref[pl.ds(rl, BM), :]

        dims = (((1,), (0,)), ((), ()))

        def bodyj(j, carry):
            rs, ws = carry
            c0 = j * BN
            t_j = txtT_ref[:, pl.ds(c0, BN)]
            i_j = imgT_ref[:, pl.ds(c0, BN)]
            sim = sim_ref[:, pl.ds(c0, BN)]
            lb = scale * lax.dot_general(a_img, t_j, dims,
                                         preferred_element_type=jnp.float32)
            ltb = scale * lax.dot_general(a_txt, i_j, dims,
                                          preferred_element_type=jnp.float32)
            c_j = idxr_ref[:, pl.ds(c0, BN)]
            w = jnp.where((sim >= thresh) & (c_i == c_j), sim, 0.0)
            rs = rs + jnp.sum(w, axis=1, keepdims=True)
            ws = ws + jnp.sum(w * (lb + ltb), axis=1, keepdims=True)
            return rs, ws

        rs, ws = lax.fori_loop(
            0, NB, bodyj,
            (jnp.zeros((BM, 1), jnp.float32),
             jnp.zeros((BM, 1), jnp.float32)))

        simd = simd_ref[pl.ds(rl, BM), :]
        lii = lii_ref[pl.ds(rl, BM), :]
        rowsum = simd + rs
        wsum = simd * 2.0 * lii + ws
        out_ref[...] = (lsei_ref[pl.ds(rl, BM), :]
                        + lset_ref[pl.ds(rl, BM), :] - wsum / rowsum)

    return body


@functools.partial(jax.jit, static_argnames=("interpret",))
def kernel(image_features, text_features, logit_scale, img_index,
           interpret=False):
    img = image_features.astype(jnp.float32)
    txt = text_features.astype(jnp.float32)
    imgT = img.T
    txtT = txt.T
    scale = jnp.asarray(logit_scale, jnp.float32).reshape(1, 1)
    idxc = img_index.astype(jnp.int32).reshape(B, 1)
    idxr = img_index.astype(jnp.int32).reshape(1, B)

    full = lambda shape: pl.BlockSpec(shape, lambda i: (0,) * len(shape))
    colspec = pl.BlockSpec((BM, 1), lambda i: (i, 0))
    iota16 = jnp.arange(NL, dtype=jnp.int32)
    mesh = plsc.VectorSubcoreMesh(core_axis_name="c", subcore_axis_name="s")

    sims, auxs, tops = [], [], []
    for q in range(NCH):
        sim, lse_i, lse_t, lii, simd = pl.pallas_call(
            _make_sim_lse(q),
            grid=(NBC,),
            in_specs=[full((B, D)), full((B, D)), full((D, B)), full((D, B)),
                      full((1, 1))],
            out_specs=[pl.BlockSpec((BM, B), lambda i: (i, 0)),
                       colspec, colspec, colspec, colspec],
            out_shape=[jax.ShapeDtypeStruct((CH, B), jnp.float32)] +
                      [jax.ShapeDtypeStruct((CH, 1), jnp.float32)] * 4,
            interpret=interpret,
        )(img, txt, imgT, txtT, scale)
        sims.append(sim)
        auxs.append((lse_i, lse_t, lii, simd))

    for q in range(NCH):
        top16 = pl.kernel(
            _topk_sc,
            out_type=jax.ShapeDtypeStruct((CH, NL), jnp.float32),
            mesh=mesh,
            scratch_types=[pltpu.VMEM((RG, B), jnp.float32),
                           pltpu.VMEM((RG, NL), jnp.float32),
                           pltpu.VMEM((NL,), jnp.int32)],
            compiler_params=pltpu.CompilerParams(needs_layout_passes=False),
            interpret=interpret,
        )(sims[q], iota16)
        tops.append(top16[:, 6:7])

    outs = []
    for q in range(NCH):
        lse_i, lse_t, lii, simd = auxs[q]
        out = pl.pallas_call(
            _make_select(q),
            grid=(NBC,),
            in_specs=[pl.BlockSpec((BM, B), lambda i: (i, 0)),
                      full((B, D)), full((B, D)), full((D, B)), full((D, B)),
                      full((B, 1)), full((1, B)), full((1, 1)),
                      full((CH, 1)), full((CH, 1)), full((CH, 1)),
                      full((CH, 1)), full((CH, 1))],
            out_specs=colspec,
            out_shape=jax.ShapeDtypeStruct((CH, 1), jnp.float32),
            interpret=interpret,
        )(sims[q], img, txt, imgT, txtT, idxc, idxr, scale, tops[q],
          lse_i, lse_t, lii, simd)
        outs.append(out)

    return 0.5 * jnp.mean(jnp.concatenate(outs, axis=0))
